# baseline (speedup 1.0000x reference)
import jax
import jax.numpy as jnp
from jax import lax
from jax.experimental import pallas as pl
from jax.experimental.pallas import tpu as pltpu

_ALPHA = 0.5
_NCH = 4


def _linear_kernel(x_hbm, w_in_ref, b_in_ref, w_out_ref, b_out_ref, o_hbm,
                   xbuf, obuf, in_sems, out_sems):
    for i in range(_NCH):
        pltpu.make_async_copy(x_hbm.at[i], xbuf.at[i], in_sems.at[i]).start()
    w = _ALPHA * w_in_ref[...] + (1.0 - _ALPHA) * w_out_ref[...]
    bcol = (_ALPHA * b_in_ref[...] + (1.0 - _ALPHA) * b_out_ref[...])[:, None]
    for i in range(_NCH):
        pltpu.make_async_copy(x_hbm.at[i], xbuf.at[i], in_sems.at[i]).wait()
        acc = lax.dot_general(
            w, xbuf[i],
            dimension_numbers=(((1,), (1,)), ((), ())),
            preferred_element_type=jnp.float32,
        )
        obuf[i] = acc + bcol
        pltpu.make_async_copy(obuf.at[i], o_hbm.at[i], out_sems.at[i]).start()
    for i in range(_NCH):
        pltpu.make_async_copy(obuf.at[i], o_hbm.at[i], out_sems.at[i]).wait()


def kernel(x, At, W_in, b_in, W_out, b_out):
    del At
    Bd, Nd, L = x.shape
    out_ch = W_in.shape[0]

    out_t = pl.pallas_call(
        _linear_kernel,
        in_specs=[
            pl.BlockSpec(memory_space=pl.ANY),
            pl.BlockSpec(memory_space=pltpu.MemorySpace.VMEM),
            pl.BlockSpec(memory_space=pltpu.MemorySpace.VMEM),
            pl.BlockSpec(memory_space=pltpu.MemorySpace.VMEM),
            pl.BlockSpec(memory_space=pltpu.MemorySpace.VMEM),
        ],
        out_specs=pl.BlockSpec(memory_space=pl.ANY),
        out_shape=jax.ShapeDtypeStruct((Bd, out_ch, Nd), jnp.float32),
        scratch_shapes=[
            pltpu.VMEM((Bd, Nd, L), jnp.float32),
            pltpu.VMEM((Bd, out_ch, Nd), jnp.float32),
            pltpu.SemaphoreType.DMA((_NCH,)),
            pltpu.SemaphoreType.DMA((_NCH,)),
        ],
    )(x, W_in, b_in, W_out, b_out)
    return out_t.transpose(0, 2, 1)


# speedup vs baseline: 1.1489x; 1.1489x over previous
import jax
import jax.numpy as jnp
from jax import lax
from jax.experimental import pallas as pl
from jax.experimental.pallas import tpu as pltpu

_ALPHA = 0.5
_B_BLOCK = 2


def _linear_kernel(x_ref, w_in_ref, b_in_ref, w_out_ref, b_out_ref, o_ref):
    w = _ALPHA * w_in_ref[...] + (1.0 - _ALPHA) * w_out_ref[...]
    b = _ALPHA * b_in_ref[...] + (1.0 - _ALPHA) * b_out_ref[...]
    bcol = b[:, None]
    for bb in range(_B_BLOCK):
        acc = lax.dot_general(
            w, x_ref[bb],
            dimension_numbers=(((1,), (1,)), ((), ())),
            preferred_element_type=jnp.float32,
        )
        o_ref[bb] = acc + bcol


def kernel(x, At, W_in, b_in, W_out, b_out):
    del At
    Bd, Nd, L = x.shape
    out_ch = W_in.shape[0]

    grid = (Bd // _B_BLOCK,)
    out_t = pl.pallas_call(
        _linear_kernel,
        grid=grid,
        in_specs=[
            pl.BlockSpec((_B_BLOCK, Nd, L), lambda i: (i, 0, 0)),
            pl.BlockSpec((out_ch, L), lambda i: (0, 0)),
            pl.BlockSpec((out_ch,), lambda i: (0,)),
            pl.BlockSpec((out_ch, L), lambda i: (0, 0)),
            pl.BlockSpec((out_ch,), lambda i: (0,)),
        ],
        out_specs=pl.BlockSpec((_B_BLOCK, out_ch, Nd), lambda i: (i, 0, 0)),
        out_shape=jax.ShapeDtypeStruct((Bd, out_ch, Nd), jnp.float32),
        compiler_params=pltpu.CompilerParams(
            skip_device_barrier=True,
            disable_bounds_checks=True,
            disable_semaphore_checks=True,
        ),
    )(x, W_in, b_in, W_out, b_out)
    return out_t.transpose(0, 2, 1)
